# baseline (speedup 1.0000x reference)
import functools
import math

import jax
import jax.numpy as jnp
from jax import lax
from jax.experimental import pallas as pl
from jax.experimental.pallas import tpu as pltpu
from jax.experimental.pallas import tpu_sc as plsc

B, C, H, W_ = 32, 96, 32, 32
N = H * W_
K = 16
INF = 1e10
BN_SCALE = 1.0 / math.sqrt(1.0 + 1e-5)
INV_SQRT2 = 1.0 / math.sqrt(2.0)

SC_G = 16
NW = 32
CH = 16


def _pair_dist(xb):
    sq = jnp.sum(xb * xb, axis=1, keepdims=True)
    g = lax.dot_general(xb, xb, (((1,), (1,)), ((), ())),
                        preferred_element_type=jnp.float32)
    d2 = sq + jnp.transpose(sq) - 2.0 * g
    rows = lax.broadcasted_iota(jnp.int32, (N, N), 0)
    cols = lax.broadcasted_iota(jnp.int32, (N, N), 1)
    return jnp.where(rows != cols, d2, INF)


def _epilogue(xb, mask, wg_ref, bg_ref, gamma_ref, beta_ref):
    h = jnp.dot(xb, wg_ref[...], preferred_element_type=jnp.float32)
    agg = jnp.dot(mask, h, preferred_element_type=jnp.float32) + h
    y = agg * (1.0 / (K + 1)) + bg_ref[...]
    y = y * (gamma_ref[...] * BN_SCALE) + beta_ref[...]
    y = y * 0.5 * (1.0 + lax.erf(y * INV_SQRT2))
    return y + xb


def _fused_kernel(x_ref, wg_ref, bg_ref, gamma_ref, beta_ref, out_ref):
    xb = x_ref[0]
    m = _pair_dist(xb)
    cur = jnp.min(m, axis=1, keepdims=True)
    for _ in range(K - 1):
        cur = jnp.min(jnp.where(m > cur, m, INF), axis=1, keepdims=True)
    mask = jnp.where(m <= cur, 1.0, 0.0)
    out_ref[0] = _epilogue(xb, mask, wg_ref, bg_ref, gamma_ref, beta_ref)


def _dist_kernel(x_ref, d2_ref):
    d2_ref[0] = _pair_dist(x_ref[0])


def _sc_sort(v, descending=False):
    return plsc.sort_key_val(v, v, descending=descending)[0]


def _sc_topk_body(d2_hbm, t_hbm, buf_v, out_v):
    rows_per_w = (SC_G * N) // NW
    nchunk = rows_per_w // CH
    wid = lax.axis_index("s") * 2 + lax.axis_index("c")
    base = wid * rows_per_w

    def chunk_body(ci, _):
        pltpu.sync_copy(d2_hbm.at[pl.ds(base + ci * CH, CH)], buf_v)
        accs = tuple(
            _sc_sort(buf_v[r, pl.ds(0, 16)]) for r in range(CH))

        def leaf_body(j, accs):
            new = []
            for r in range(CH):
                leaf = _sc_sort(buf_v[r, pl.ds(j * 16, 16)], descending=True)
                low = jnp.minimum(accs[r], leaf)
                new.append(_sc_sort(low))
            return tuple(new)

        accs = lax.fori_loop(1, N // 16, leaf_body, accs)
        lanes = lax.broadcasted_iota(jnp.int32, (16,), 0)
        tvec = jnp.zeros((16,), jnp.float32)
        for r in range(CH):
            tr = jnp.max(accs[r])
            tvec = jnp.where(lanes == r, jnp.full((16,), tr), tvec)
        out_v[pl.ds(ci * CH, CH)] = tvec
        return 0

    lax.fori_loop(0, nchunk, chunk_body, 0)
    pltpu.sync_copy(out_v, t_hbm.at[pl.ds(base, rows_per_w)])


def _final_kernel(x_ref, t_ref, wg_ref, bg_ref, gamma_ref, beta_ref, out_ref):
    xb = x_ref[0]
    m = _pair_dist(xb)
    mask = jnp.where(m <= t_ref[0], 1.0, 0.0)
    out_ref[0] = _epilogue(xb, mask, wg_ref, bg_ref, gamma_ref, beta_ref)


@jax.jit
def kernel(x, Wg, bg, gamma, beta):
    x_nodes = jnp.transpose(x, (0, 2, 3, 1)).reshape(B, N, C)
    bg2 = bg.reshape(1, C)
    ga2 = gamma.reshape(1, C)
    be2 = beta.reshape(1, C)
    xb_sc = x_nodes[:SC_G]
    xb_tc = x_nodes[SC_G:]

    d2 = pl.pallas_call(
        _dist_kernel,
        grid=(SC_G,),
        in_specs=[pl.BlockSpec((1, N, C), lambda b: (b, 0, 0))],
        out_specs=pl.BlockSpec((1, N, N), lambda b: (b, 0, 0)),
        out_shape=jax.ShapeDtypeStruct((SC_G, N, N), jnp.float32),
    )(xb_sc).reshape(SC_G * N, N)

    mesh = plsc.VectorSubcoreMesh(core_axis_name="c", subcore_axis_name="s")
    t = pl.kernel(
        _sc_topk_body,
        out_type=jax.ShapeDtypeStruct((SC_G * N,), jnp.float32),
        mesh=mesh,
        scratch_types=[
            pltpu.VMEM((CH, N), jnp.float32),
            pltpu.VMEM(((SC_G * N) // NW,), jnp.float32),
        ],
        compiler_params=pltpu.CompilerParams(needs_layout_passes=False),
    )(d2)

    out_tc = pl.pallas_call(
        _fused_kernel,
        grid=(B - SC_G,),
        in_specs=[
            pl.BlockSpec((1, N, C), lambda b: (b, 0, 0)),
            pl.BlockSpec((C, C), lambda b: (0, 0)),
            pl.BlockSpec((1, C), lambda b: (0, 0)),
            pl.BlockSpec((1, C), lambda b: (0, 0)),
            pl.BlockSpec((1, C), lambda b: (0, 0)),
        ],
        out_specs=pl.BlockSpec((1, N, C), lambda b: (b, 0, 0)),
        out_shape=jax.ShapeDtypeStruct((B - SC_G, N, C), jnp.float32),
    )(xb_tc, Wg, bg2, ga2, be2)

    out_sc = pl.pallas_call(
        _final_kernel,
        grid=(SC_G,),
        in_specs=[
            pl.BlockSpec((1, N, C), lambda b: (b, 0, 0)),
            pl.BlockSpec((1, N, 1), lambda b: (b, 0, 0)),
            pl.BlockSpec((C, C), lambda b: (0, 0)),
            pl.BlockSpec((1, C), lambda b: (0, 0)),
            pl.BlockSpec((1, C), lambda b: (0, 0)),
            pl.BlockSpec((1, C), lambda b: (0, 0)),
        ],
        out_specs=pl.BlockSpec((1, N, C), lambda b: (b, 0, 0)),
        out_shape=jax.ShapeDtypeStruct((SC_G, N, C), jnp.float32),
    )(xb_sc, t.reshape(SC_G, N, 1), Wg, bg2, ga2, be2)

    out = jnp.concatenate([out_sc, out_tc], axis=0)
    return out.reshape(B, H, W_, C).transpose(0, 3, 1, 2)


# speedup vs baseline: 1.1065x; 1.1065x over previous
import functools
import math

import jax
import jax.numpy as jnp
from jax import lax
from jax.experimental import pallas as pl
from jax.experimental.pallas import tpu as pltpu
from jax.experimental.pallas import tpu_sc as plsc

B, C, H, W_ = 32, 96, 32, 32
N = H * W_
K = 16
INF = 1e10
BN_SCALE = 1.0 / math.sqrt(1.0 + 1e-5)
INV_SQRT2 = 1.0 / math.sqrt(2.0)

SC_G = 16
NW = 32
CH = 16


def _pair_dist(xcb):
    sq = jnp.sum(xcb * xcb, axis=0, keepdims=True)
    g = lax.dot_general(xcb, xcb, (((0,), (0,)), ((), ())),
                        preferred_element_type=jnp.float32)
    d2 = jnp.transpose(sq) + sq - 2.0 * g
    rows = lax.broadcasted_iota(jnp.int32, (N, N), 0)
    cols = lax.broadcasted_iota(jnp.int32, (N, N), 1)
    return jnp.where(rows != cols, d2, INF)


def _epilogue(xcb, mask, wg_ref, bg_ref, gamma_ref, beta_ref):
    ht = lax.dot_general(wg_ref[...], xcb, (((0,), (0,)), ((), ())),
                         preferred_element_type=jnp.float32)
    agg = lax.dot_general(ht, mask, (((1,), (0,)), ((), ())),
                          preferred_element_type=jnp.float32) + ht
    y = agg * (1.0 / (K + 1)) + bg_ref[...]
    y = y * (gamma_ref[...] * BN_SCALE) + beta_ref[...]
    y = y * 0.5 * (1.0 + lax.erf(y * INV_SQRT2))
    return y + xcb


def _fused_kernel(x_ref, wg_ref, bg_ref, gamma_ref, beta_ref, out_ref):
    xcb = x_ref[0]
    m = _pair_dist(xcb)
    cur = jnp.min(m, axis=1, keepdims=True)
    for _ in range(K - 1):
        cur = jnp.min(jnp.where(m > cur, m, INF), axis=1, keepdims=True)
    mask = jnp.where(m <= jnp.transpose(cur), 1.0, 0.0)
    out_ref[0] = _epilogue(xcb, mask, wg_ref, bg_ref, gamma_ref, beta_ref)


def _dist_kernel(x_ref, d2_ref):
    d2_ref[0] = _pair_dist(x_ref[0])


def _sc_sort(v, descending=False):
    return plsc.sort_key_val(v, v, descending=descending)[0]


def _sc_topk_body(d2_hbm, t_hbm, buf_v, out_v):
    rows_per_w = (SC_G * N) // NW
    nchunk = rows_per_w // CH
    wid = lax.axis_index("s") * 2 + lax.axis_index("c")
    base = wid * rows_per_w

    def chunk_body(ci, _):
        pltpu.sync_copy(d2_hbm.at[pl.ds(base + ci * CH, CH)], buf_v)
        accs = tuple(
            _sc_sort(buf_v[r, pl.ds(0, 16)]) for r in range(CH))

        def leaf_body(j, accs):
            new = []
            for r in range(CH):
                leaf = _sc_sort(buf_v[r, pl.ds(j * 16, 16)], descending=True)
                low = jnp.minimum(accs[r], leaf)
                new.append(_sc_sort(low))
            return tuple(new)

        accs = lax.fori_loop(1, N // 16, leaf_body, accs)
        lanes = lax.broadcasted_iota(jnp.int32, (16,), 0)
        tvec = jnp.zeros((16,), jnp.float32)
        for r in range(CH):
            tr = jnp.max(accs[r])
            tvec = jnp.where(lanes == r, jnp.full((16,), tr), tvec)
        out_v[pl.ds(ci * CH, CH)] = tvec
        return 0

    lax.fori_loop(0, nchunk, chunk_body, 0)
    pltpu.sync_copy(out_v, t_hbm.at[pl.ds(base, rows_per_w)])


def _final_kernel(buf_ref, x_ref, t_ref, wg_ref, bg_ref, gamma_ref, beta_ref,
                  out_ref):
    del buf_ref
    xcb = x_ref[0]
    m = _pair_dist(xcb)
    mask = jnp.where(m <= t_ref[0], 1.0, 0.0)
    out_ref[0] = _epilogue(xcb, mask, wg_ref, bg_ref, gamma_ref, beta_ref)


@jax.jit
def kernel(x, Wg, bg, gamma, beta):
    xr = x.reshape(B, C, N)
    bgc = bg.reshape(C, 1)
    gac = gamma.reshape(C, 1)
    bec = beta.reshape(C, 1)

    d2 = pl.pallas_call(
        _dist_kernel,
        grid=(SC_G,),
        in_specs=[pl.BlockSpec((1, C, N), lambda b: (b, 0, 0))],
        out_specs=pl.BlockSpec((1, N, N), lambda b: (b, 0, 0)),
        out_shape=jax.ShapeDtypeStruct((SC_G, N, N), jnp.float32),
    )(xr).reshape(SC_G * N, N)

    mesh = plsc.VectorSubcoreMesh(core_axis_name="c", subcore_axis_name="s")
    t = pl.kernel(
        _sc_topk_body,
        out_type=jax.ShapeDtypeStruct((SC_G * N,), jnp.float32),
        mesh=mesh,
        scratch_types=[
            pltpu.VMEM((CH, N), jnp.float32),
            pltpu.VMEM(((SC_G * N) // NW,), jnp.float32),
        ],
        compiler_params=pltpu.CompilerParams(needs_layout_passes=False),
    )(d2)

    buf = pl.pallas_call(
        _fused_kernel,
        grid=(B - SC_G,),
        in_specs=[
            pl.BlockSpec((1, C, N), lambda b: (b + SC_G, 0, 0)),
            pl.BlockSpec((C, C), lambda b: (0, 0)),
            pl.BlockSpec((C, 1), lambda b: (0, 0)),
            pl.BlockSpec((C, 1), lambda b: (0, 0)),
            pl.BlockSpec((C, 1), lambda b: (0, 0)),
        ],
        out_specs=pl.BlockSpec((1, C, N), lambda b: (b + SC_G, 0, 0)),
        out_shape=jax.ShapeDtypeStruct((B, C, N), jnp.float32),
    )(xr, Wg, bgc, gac, bec)

    out = pl.pallas_call(
        _final_kernel,
        grid=(SC_G,),
        in_specs=[
            pl.BlockSpec((1, C, N), lambda b: (b, 0, 0)),
            pl.BlockSpec((1, C, N), lambda b: (b, 0, 0)),
            pl.BlockSpec((1, 1, N), lambda b: (b, 0, 0)),
            pl.BlockSpec((C, C), lambda b: (0, 0)),
            pl.BlockSpec((C, 1), lambda b: (0, 0)),
            pl.BlockSpec((C, 1), lambda b: (0, 0)),
            pl.BlockSpec((C, 1), lambda b: (0, 0)),
        ],
        out_specs=pl.BlockSpec((1, C, N), lambda b: (b, 0, 0)),
        out_shape=jax.ShapeDtypeStruct((B, C, N), jnp.float32),
        input_output_aliases={0: 0},
    )(buf, xr, t.reshape(SC_G, 1, N), Wg, bgc, gac, bec)

    return out.reshape(B, C, H, W_)
